# baseline (speedup 1.0000x reference)
import functools
import math

import jax
import jax.numpy as jnp
import numpy as np
from jax import lax
from jax.experimental import pallas as pl
from jax.experimental.pallas import tpu as pltpu
from jax.experimental.pallas import tpu_sc as plsc

N_NODES = 10000
N_EDGES = 320000
D = 128
D_ATTR = 16
D_EF = 16
DW = D // 2

NC = 2
NS = 16
NW = NC * NS
E_PER_W = N_EDGES // NW
K = 80
NCHUNK = E_PER_W // K
N_PAD = 10240
ROWS_PER_TILE = N_PAD // NS
ZROWS = 16

_PERM = np.zeros(D, dtype=np.int32)
for _d4 in range(4):
    for _i in range(16):
        _PERM[16 * _d4 + _i] = 32 * _d4 + _i
        _PERM[64 + 16 * _d4 + _i] = 32 * _d4 + 16 + _i


def _pack_rows_u32(x_perm_f32):
    xb = x_perm_f32.astype(jnp.bfloat16)
    u = lax.bitcast_convert_type(xb, jnp.uint16).astype(jnp.uint32)
    return u[:, :DW] | (u[:, DW:] << 16)


EBLK2 = 2000


def _round_pack(c_f32):
    u = lax.bitcast_convert_type(c_f32, jnp.uint32)
    r = (u + 0x7FFF + ((u >> 16) & 1)) >> 16
    return r[:, :DW] | (r[:, DW:] << 16)


def _coef_body(ef2_ref, ea2_ref, wa_ref, wb_ref, c_ref):
    s = 1.0 / math.sqrt(D_EF)
    ce = jnp.dot(ef2_ref[...], wa_ref[...], preferred_element_type=jnp.float32)
    co = jnp.dot(ef2_ref[...], wb_ref[...], preferred_element_type=jnp.float32)
    ce = ce * (ea2_ref[:, 0:1] * s)
    co = co * (ea2_ref[:, 1:2] * s)
    c_ref[...] = jnp.concatenate([_round_pack(ce), _round_pack(co)], axis=1)


def _edge_coefs(ef2, ea2, W2a, W2b):
    return pl.pallas_call(
        _coef_body,
        grid=(N_EDGES // 2 // EBLK2,),
        in_specs=[
            pl.BlockSpec((EBLK2, 2 * D_EF), lambda i: (i, 0)),
            pl.BlockSpec((EBLK2, 2), lambda i: (i, 0)),
            pl.BlockSpec((2 * D_EF, D), lambda i: (0, 0)),
            pl.BlockSpec((2 * D_EF, D), lambda i: (0, 0)),
        ],
        out_specs=pl.BlockSpec((EBLK2, D), lambda i: (i, 0)),
        out_shape=jax.ShapeDtypeStruct((N_EDGES // 2, D), jnp.uint32),
    )(ef2, ea2, W2a, W2b)


def _sc_body(nf_hbm, c_hbm, snd_hbm, rcv_hbm, out_hbm,
             snd_v, rcv_v, buf_v, c_v, zbuf_v, msg_sh,
             isem0, isem1, isem2, isem3, gsem0, gsem1, gsem2,
             csem0, csem1, ssem0, ssem1, ssem2):
    cid = lax.axis_index("c")
    sid = lax.axis_index("s")
    wid = sid * NC + cid
    isems = (isem0, isem1, isem2, isem3)
    gsems = (gsem0, gsem1, gsem2)
    csems = (csem0, csem1)
    ssems = (ssem0, ssem1, ssem2)
    ebase = wid * E_PER_W
    cbase = wid * (E_PER_W // 2)

    def _start_idx(j, q):
        b = ebase + j * K
        pltpu.async_copy(snd_hbm.at[pl.ds(b, K)], snd_v.at[q], isems[q])
        pltpu.async_copy(rcv_hbm.at[pl.ds(b, K)], rcv_v.at[q], isems[q])

    def _wait_idx(j, q):
        b = ebase + j * K
        pltpu.make_async_copy(snd_hbm.at[pl.ds(b, K)], snd_v.at[q], isems[q]).wait()
        pltpu.make_async_copy(rcv_hbm.at[pl.ds(b, K)], rcv_v.at[q], isems[q]).wait()

    def _start_data(j, q, t, p):
        pltpu.async_copy(nf_hbm.at[snd_v.at[q]], buf_v.at[t], gsems[t])
        pltpu.async_copy(c_hbm.at[pl.ds(cbase + j * (K // 2), K // 2)], c_v.at[p], csems[p])

    def _wait_data(j, q, t, p):
        pltpu.make_async_copy(nf_hbm.at[snd_v.at[q]], buf_v.at[t], gsems[t]).wait()
        pltpu.make_async_copy(c_hbm.at[pl.ds(cbase + j * (K // 2), K // 2)], c_v.at[p], csems[p]).wait()

    def _start_scat(q, t):
        pltpu.async_copy(buf_v.at[t], msg_sh.at[rcv_v.at[q]], ssems[t], add=True)

    def _wait_scat(q, t):
        pltpu.make_async_copy(buf_v.at[t], msg_sh.at[rcv_v.at[q]], ssems[t]).wait()

    _start_idx(0, 0)
    _wait_idx(0, 0)
    _start_data(0, 0, 0, 0)
    _start_idx(1, 1)

    def _zrow(i, carry):
        for d8 in range(D // 16):
            zbuf_v[i, pl.ds(d8 * 16, 16)] = jnp.zeros((16,), jnp.float32)
        return carry

    lax.fori_loop(0, ZROWS, _zrow, 0)
    for j in range(ROWS_PER_TILE // ZROWS):
        pltpu.sync_copy(zbuf_v, msg_sh.at[pl.ds(sid * ROWS_PER_TILE + j * ZROWS, ZROWS)])
    plsc.subcore_barrier()

    def _step(jj, r):
        p = r % 2
        t = r % 3
        q = r % 4
        t1 = (r + 1) % 3
        q1 = (r + 1) % 4
        q2 = (r + 2) % 4

        @pl.when(jj < NCHUNK)
        def _():
            @pl.when(jj >= 2)
            def _():
                _wait_scat(q2, t1)

            @pl.when(jj + 1 < NCHUNK)
            def _():
                _wait_idx(jj + 1, q1)
                _start_data(jj + 1, q1, t1, 1 - p)

            _wait_data(jj, q, t, p)

            def _mulrow(e2, c2):
                hi_mask = jnp.uint32(0xFFFF0000)
                sh = jnp.uint32(16)
                for h in range(2):
                    e = 2 * e2 + h
                    for d4 in range(4):
                        cw = c_v[p, e2, pl.ds(h * DW + d4 * 16, 16)]
                        ca = lax.bitcast_convert_type(cw << sh, jnp.float32)
                        cb = lax.bitcast_convert_type(cw & hi_mask, jnp.float32)
                        sla = pl.ds(d4 * 32, 16)
                        slb = pl.ds(d4 * 32 + 16, 16)
                        buf_v[t, e, sla] = buf_v[t, e, sla] * ca
                        buf_v[t, e, slb] = buf_v[t, e, slb] * cb
                return c2

            lax.fori_loop(0, K // 2, _mulrow, 0)
            _start_scat(q, t)

            @pl.when(jj + 2 < NCHUNK)
            def _():
                _start_idx(jj + 2, q2)

    def _block(jb, carry):
        for r in range(12):
            _step(12 * jb + r, r)
        return carry

    lax.fori_loop(0, (NCHUNK + 11) // 12, _block, 0)
    _wait_scat(3, 0)
    _wait_scat(0, 1)
    plsc.subcore_barrier()

    for j in range(ROWS_PER_TILE // ZROWS):
        r0 = sid * ROWS_PER_TILE + j * ZROWS
        pltpu.sync_copy(msg_sh.at[pl.ds(r0, ZROWS)], out_hbm.at[cid, pl.ds(r0, ZROWS)])


def _sc_message(nf_packed, c_packed, snd, rcv):
    mesh = plsc.VectorSubcoreMesh(core_axis_name="c", subcore_axis_name="s")
    f = functools.partial(
        pl.kernel,
        out_type=jax.ShapeDtypeStruct((NC, N_PAD, D), jnp.float32),
        mesh=mesh,
        scratch_types=[
            pltpu.VMEM((4, K), jnp.int32),
            pltpu.VMEM((4, K), jnp.int32),
            pltpu.VMEM((3, K, D), jnp.float32),
            pltpu.VMEM((2, K // 2, D), jnp.uint32),
            pltpu.VMEM((ZROWS, D), jnp.float32),
            pltpu.MemorySpace.VMEM_SHARED((N_PAD, D), jnp.float32),
        ] + [pltpu.SemaphoreType.DMA] * 12,
    )(_sc_body)
    return f(nf_packed, c_packed, snd, rcv)


NBLK = 2000


def _final_body(p_ref, attrs_ref, wl_ref, wst_ref, out_ref):
    msg = p_ref[0] + p_ref[1]
    m2 = jnp.dot(msg, wl_ref[...], preferred_element_type=jnp.float32) * (
        1.0 / math.sqrt(D))
    acc = jnp.zeros((NBLK, D), jnp.float32)
    for v in range(D_ATTR):
        acc = acc + attrs_ref[:, v:v + 1] * jnp.dot(
            m2, wst_ref[v], preferred_element_type=jnp.float32)
    out_ref[...] = acc * (1.0 / math.sqrt(D * D_ATTR))


def _final(partials, node_attrs, W_linear, W_skip_t):
    return pl.pallas_call(
        _final_body,
        grid=(N_NODES // NBLK,),
        in_specs=[
            pl.BlockSpec((NC, NBLK, D), lambda i: (0, i, 0)),
            pl.BlockSpec((NBLK, D_ATTR), lambda i: (i, 0)),
            pl.BlockSpec((D, D), lambda i: (0, 0)),
            pl.BlockSpec((D_ATTR, D, D), lambda i: (0, 0, 0)),
        ],
        out_specs=pl.BlockSpec((NBLK, D), lambda i: (i, 0)),
        out_shape=jax.ShapeDtypeStruct((N_NODES, D), jnp.float32),
    )(partials, node_attrs, W_linear, W_skip_t)


def kernel(node_attrs, node_feats, edge_attrs, edge_feats, edge_index, W_radial, W_linear, W_skip):
    snd = edge_index[0]
    rcv = edge_index[1]
    perm = jnp.asarray(_PERM)
    Wp = W_radial[:, perm]
    zeros = jnp.zeros_like(Wp)
    W2a = jnp.concatenate([Wp, zeros], axis=0)
    W2b = jnp.concatenate([zeros, Wp], axis=0)
    ef2 = edge_feats.reshape(N_EDGES // 2, 2 * D_EF)
    ea2 = edge_attrs.reshape(N_EDGES // 2, 2)
    c_packed = _edge_coefs(ef2, ea2, W2a, W2b)
    partials = _sc_message(node_feats, c_packed, snd, rcv)
    return _final(partials, node_attrs, W_linear, W_skip.transpose(1, 0, 2))


# speedup vs baseline: 1.3019x; 1.3019x over previous
import functools
import math

import jax
import jax.numpy as jnp
import numpy as np
from jax import lax
from jax.experimental import pallas as pl
from jax.experimental.pallas import tpu as pltpu
from jax.experimental.pallas import tpu_sc as plsc

N_NODES = 10000
N_EDGES = 320000
D = 128
D_ATTR = 16
D_EF = 16
DW = D // 2

NC = 2
NS = 16
NW = NC * NS
E_PER_W = N_EDGES // NW
K = 80
NCHUNK = E_PER_W // K
N_PAD = 10240
ROWS_PER_TILE = N_PAD // NS
ZROWS = 32

_PERM = np.zeros(D, dtype=np.int32)
for _d4 in range(4):
    for _i in range(16):
        _PERM[16 * _d4 + _i] = 32 * _d4 + _i
        _PERM[64 + 16 * _d4 + _i] = 32 * _d4 + 16 + _i
_PMAT = np.zeros((D, D), dtype=np.float32)
for _j in range(D):
    _PMAT[_PERM[_j], _j] = 1.0


EBLK = 4000


def _coef_body(ef_ref, ea_ref, wr_ref, pm_ref, c_ref):
    s = 1.0 / math.sqrt(D_EF)
    cf = jnp.dot(ef_ref[...], wr_ref[...], preferred_element_type=jnp.float32)
    cf = jnp.dot(cf * (ea_ref[...] * s), pm_ref[...],
                 preferred_element_type=jnp.float32)
    u = lax.bitcast_convert_type(cf, jnp.uint32)
    r = (u + 0x7FFF + ((u >> 16) & 1)) >> 16
    c_ref[...] = r[:, :DW] | (r[:, DW:] << 16)


def _edge_coefs(edge_feats, edge_attrs, W_radial, pmat):
    return pl.pallas_call(
        _coef_body,
        grid=(N_EDGES // EBLK,),
        in_specs=[
            pl.BlockSpec((EBLK, D_EF), lambda i: (i, 0)),
            pl.BlockSpec((EBLK, 1), lambda i: (i, 0)),
            pl.BlockSpec((D_EF, D), lambda i: (0, 0)),
            pl.BlockSpec((D, D), lambda i: (0, 0)),
        ],
        out_specs=pl.BlockSpec((EBLK, DW), lambda i: (i, 0)),
        out_shape=jax.ShapeDtypeStruct((N_EDGES, DW), jnp.uint32),
    )(edge_feats, edge_attrs, W_radial, pmat)


def _sc_body(nf_hbm, c_hbm, snd_hbm, rcv_hbm, out_hbm,
             snd_v, rcv_v, rows_v, c_v, zbuf_v, msg_sh,
             isem0, isem1, gsem0, gsem1, csem0, csem1):
    cid = lax.axis_index("c")
    sid = lax.axis_index("s")
    wid = sid * NC + cid
    isems = (isem0, isem1)
    gsems = (gsem0, gsem1)
    csems = (csem0, csem1)
    ebase = wid * E_PER_W

    def _start_idx(j, q):
        b = ebase + j * K
        pltpu.async_copy(snd_hbm.at[pl.ds(b, K)], snd_v.at[q], isems[q])
        pltpu.async_copy(rcv_hbm.at[pl.ds(b, K)], rcv_v.at[q], isems[q])

    def _wait_idx(j, q):
        b = ebase + j * K
        pltpu.make_async_copy(snd_hbm.at[pl.ds(b, K)], snd_v.at[q], isems[q]).wait()
        pltpu.make_async_copy(rcv_hbm.at[pl.ds(b, K)], rcv_v.at[q], isems[q]).wait()

    def _start_data(j, q, p):
        pltpu.async_copy(nf_hbm.at[snd_v.at[q]], rows_v.at[p], gsems[p])
        pltpu.async_copy(c_hbm.at[pl.ds(ebase + j * K, K)], c_v.at[p], csems[p])

    def _wait_data(j, q, p):
        pltpu.make_async_copy(nf_hbm.at[snd_v.at[q]], rows_v.at[p], gsems[p]).wait()
        pltpu.make_async_copy(c_hbm.at[pl.ds(ebase + j * K, K)], c_v.at[p], csems[p]).wait()

    _start_idx(0, 0)
    _wait_idx(0, 0)
    _start_data(0, 0, 0)
    _start_idx(1, 1)

    def _zrow(i, carry):
        for d8 in range(D // 16):
            zbuf_v[i, pl.ds(d8 * 16, 16)] = jnp.zeros((16,), jnp.float32)
        return carry

    lax.fori_loop(0, ZROWS, _zrow, 0)
    for j in range(ROWS_PER_TILE // ZROWS):
        pltpu.sync_copy(zbuf_v, msg_sh.at[pl.ds(sid * ROWS_PER_TILE + j * ZROWS, ZROWS)])
    plsc.subcore_barrier()

    def _step(jj, p):
        @pl.when(jj < NCHUNK)
        def _():
            q1 = 1 - p

            @pl.when(jj + 1 < NCHUNK)
            def _():
                _wait_idx(jj + 1, q1)
                _start_data(jj + 1, q1, 1 - p)

            _wait_data(jj, p, p)

            def _mulrow(e, c2):
                hi_mask = jnp.uint32(0xFFFF0000)
                sh = jnp.uint32(16)
                for d4 in range(4):
                    cw = c_v[p, e, pl.ds(d4 * 16, 16)]
                    ca = lax.bitcast_convert_type(cw << sh, jnp.float32)
                    cb = lax.bitcast_convert_type(cw & hi_mask, jnp.float32)
                    sla = pl.ds(d4 * 32, 16)
                    slb = pl.ds(d4 * 32 + 16, 16)
                    rows_v[p, e, sla] = rows_v[p, e, sla] * ca
                    rows_v[p, e, slb] = rows_v[p, e, slb] * cb
                return c2

            lax.fori_loop(0, K, _mulrow, 0)
            pltpu.sync_copy(rows_v.at[p], msg_sh.at[rcv_v.at[p]], add=True)

            @pl.when(jj + 2 < NCHUNK)
            def _():
                _start_idx(jj + 2, p)

    def _pair(j2, carry):
        _step(2 * j2, 0)
        _step(2 * j2 + 1, 1)
        return carry

    lax.fori_loop(0, (NCHUNK + 1) // 2, _pair, 0)
    plsc.subcore_barrier()

    for j in range(ROWS_PER_TILE // ZROWS):
        r0 = sid * ROWS_PER_TILE + j * ZROWS
        pltpu.sync_copy(msg_sh.at[pl.ds(r0, ZROWS)], out_hbm.at[cid, pl.ds(r0, ZROWS)])


def _sc_message(node_feats, c_packed, snd, rcv):
    mesh = plsc.VectorSubcoreMesh(core_axis_name="c", subcore_axis_name="s")
    f = functools.partial(
        pl.kernel,
        out_type=jax.ShapeDtypeStruct((NC, N_PAD, D), jnp.float32),
        mesh=mesh,
        scratch_types=[
            pltpu.VMEM((2, K), jnp.int32),
            pltpu.VMEM((2, K), jnp.int32),
            pltpu.VMEM((2, K, D), jnp.float32),
            pltpu.VMEM((2, K, DW), jnp.uint32),
            pltpu.VMEM((ZROWS, D), jnp.float32),
            pltpu.MemorySpace.VMEM_SHARED((N_PAD, D), jnp.float32),
        ] + [pltpu.SemaphoreType.DMA] * 6,
    )(_sc_body)
    return f(node_feats, c_packed, snd, rcv)


NBLK = 2000


def _final_body(p_ref, attrs_ref, wl_ref, wst_ref, out_ref):
    msg = p_ref[0] + p_ref[1]
    m2 = jnp.dot(msg, wl_ref[...], preferred_element_type=jnp.float32) * (
        1.0 / math.sqrt(D))
    acc = jnp.zeros((NBLK, D), jnp.float32)
    for v in range(D_ATTR):
        acc = acc + attrs_ref[:, v:v + 1] * jnp.dot(
            m2, wst_ref[v], preferred_element_type=jnp.float32)
    out_ref[...] = acc * (1.0 / math.sqrt(D * D_ATTR))


def _final(partials, node_attrs, W_linear, W_skip_t):
    return pl.pallas_call(
        _final_body,
        grid=(N_NODES // NBLK,),
        in_specs=[
            pl.BlockSpec((NC, NBLK, D), lambda i: (0, i, 0)),
            pl.BlockSpec((NBLK, D_ATTR), lambda i: (i, 0)),
            pl.BlockSpec((D, D), lambda i: (0, 0)),
            pl.BlockSpec((D_ATTR, D, D), lambda i: (0, 0, 0)),
        ],
        out_specs=pl.BlockSpec((NBLK, D), lambda i: (i, 0)),
        out_shape=jax.ShapeDtypeStruct((N_NODES, D), jnp.float32),
    )(partials, node_attrs, W_linear, W_skip_t)


def kernel(node_attrs, node_feats, edge_attrs, edge_feats, edge_index, W_radial, W_linear, W_skip):
    snd = edge_index[0]
    rcv = edge_index[1]
    pmat = jnp.asarray(_PMAT)
    c_packed = _edge_coefs(edge_feats, edge_attrs, W_radial, pmat)
    partials = _sc_message(node_feats, c_packed, snd, rcv)
    return _final(partials, node_attrs, W_linear, W_skip.transpose(1, 0, 2))
